# Initial kernel scaffold; baseline (speedup 1.0000x reference)
import jax
import jax.numpy as jnp
from jax.experimental import pallas as pl


def kernel(output, y):
    raise NotImplementedError("write your pallas kernel here")


# speedup vs baseline: 1.5181x; 1.5181x over previous
import functools

import jax
import jax.numpy as jnp
from jax import lax
from jax.experimental import pallas as pl
from jax.experimental.pallas import tpu as pltpu
from jax.experimental.pallas import tpu_sc as plsc

B = 16384
C = 1000
MARGIN = 1.0

NC = 2
NS = 16
L = 16
NW = NC * NS
BPW = B // NW


def _sc_gather_body(flat_hbm, y_hbm, oy_hbm, yv, idxv, rowsv, sem):
    wid = lax.axis_index("s") * NC + lax.axis_index("c")
    base = wid * BPW
    pltpu.sync_copy(y_hbm.at[pl.ds(base, BPW)], yv)
    for j in range(BPW // L):
        row0 = base + j * L
        rows16 = row0 + lax.iota(jnp.int32, L)
        idxv[pl.ds(j * L, L)] = rows16 * C + yv[pl.ds(j * L, L)]
    pltpu.async_copy(flat_hbm.at[idxv], rowsv, sem).wait()
    pltpu.sync_copy(rowsv, oy_hbm.at[pl.ds(base, BPW)])


@functools.cache
def _sc_gather():
    return pl.kernel(
        _sc_gather_body,
        out_type=jax.ShapeDtypeStruct((B,), jnp.float32),
        mesh=plsc.VectorSubcoreMesh(core_axis_name="c", subcore_axis_name="s",
                                    num_cores=NC, num_subcores=NS),
        scratch_types=[
            pltpu.VMEM((BPW,), jnp.int32),
            pltpu.VMEM((BPW,), jnp.int32),
            pltpu.VMEM((BPW,), jnp.float32),
            pltpu.SemaphoreType.DMA,
        ],
    )


BR = 512
GRID = B // BR


def _tc_hinge_body(x_ref, oy_ref, out_ref):
    pi = pl.program_id(0)
    x = x_ref[...]
    oy = oy_ref[...]
    s = jnp.sum(jnp.maximum(x - oy + MARGIN, 0.0))

    @pl.when(pi == 0)
    def _init():
        out_ref[0, 0] = 0.0

    out_ref[0, 0] += s

    @pl.when(pi == GRID - 1)
    def _final():
        out_ref[0, 0] = (out_ref[0, 0] - float(B)) / float(B)


_tc_hinge = pl.pallas_call(
    _tc_hinge_body,
    grid=(GRID,),
    in_specs=[
        pl.BlockSpec((BR, C), lambda i: (i, 0)),
        pl.BlockSpec((BR, 1), lambda i: (i, 0)),
    ],
    out_specs=pl.BlockSpec((1, 1), lambda i: (0, 0), memory_space=pltpu.SMEM),
    out_shape=jax.ShapeDtypeStruct((1, 1), jnp.float32),
)


def kernel(output, y):
    y32 = y.astype(jnp.int32)
    oy = _sc_gather()(output.reshape(-1), y32)
    total = _tc_hinge(output, oy.reshape(B, 1))
    return total[0, 0]


# speedup vs baseline: 2.9867x; 1.9674x over previous
import functools

import jax
import jax.numpy as jnp
from jax import lax
from jax.experimental import pallas as pl
from jax.experimental.pallas import tpu as pltpu

B = 16384
C = 1000
MARGIN = 1.0

BR = 512
GRID = B // BR


def _tc_hinge_body(x_ref, y_ref, out_ref):
    pi = pl.program_id(0)
    x = x_ref[...]
    yv = y_ref[0, 0, :]
    ycol = yv.reshape(BR, 1)
    col = lax.broadcasted_iota(jnp.int32, (BR, C), 1)
    oy = jnp.sum(jnp.where(col == ycol, x, 0.0), axis=1, keepdims=True)
    s = jnp.sum(jnp.maximum(x - oy + MARGIN, 0.0))

    @pl.when(pi == 0)
    def _init():
        out_ref[0, 0] = 0.0

    out_ref[0, 0] += s

    @pl.when(pi == GRID - 1)
    def _final():
        out_ref[0, 0] = (out_ref[0, 0] - float(B)) / float(B)


_tc_hinge = pl.pallas_call(
    _tc_hinge_body,
    grid=(GRID,),
    in_specs=[
        pl.BlockSpec((BR, C), lambda i: (i, 0)),
        pl.BlockSpec((1, 1, BR), lambda i: (i, 0, 0)),
    ],
    out_specs=pl.BlockSpec((1, 1), lambda i: (0, 0), memory_space=pltpu.SMEM),
    out_shape=jax.ShapeDtypeStruct((1, 1), jnp.float32),
)


def kernel(output, y):
    y3 = y.astype(jnp.int32).reshape(GRID, 1, BR)
    total = _tc_hinge(output, y3)
    return total[0, 0]


# speedup vs baseline: 3.2610x; 1.0919x over previous
import functools

import jax
import jax.numpy as jnp
from jax import lax
from jax.experimental import pallas as pl
from jax.experimental.pallas import tpu as pltpu

B = 16384
C = 1000
MARGIN = 1.0

BR = 1024
GRID = B // BR


def _tc_hinge_body(x_ref, y_ref, out_ref):
    pi = pl.program_id(0)
    x = x_ref[...]
    yv = y_ref[0, 0, :]
    ycol = yv.reshape(BR, 1)
    col = lax.broadcasted_iota(jnp.int32, (BR, C), 1)
    oy = jnp.sum(jnp.where(col == ycol, x, 0.0), axis=1, keepdims=True)
    s = jnp.sum(jnp.maximum(x - oy + MARGIN, 0.0))

    @pl.when(pi == 0)
    def _init():
        out_ref[0, 0] = 0.0

    out_ref[0, 0] += s

    @pl.when(pi == GRID - 1)
    def _final():
        out_ref[0, 0] = (out_ref[0, 0] - float(B)) / float(B)


_tc_hinge = pl.pallas_call(
    _tc_hinge_body,
    grid=(GRID,),
    in_specs=[
        pl.BlockSpec((BR, C), lambda i: (i, 0)),
        pl.BlockSpec((1, 1, BR), lambda i: (i, 0, 0)),
    ],
    out_specs=pl.BlockSpec((1, 1), lambda i: (0, 0), memory_space=pltpu.SMEM),
    out_shape=jax.ShapeDtypeStruct((1, 1), jnp.float32),
)


def kernel(output, y):
    y3 = y.astype(jnp.int32).reshape(GRID, 1, BR)
    total = _tc_hinge(output, y3)
    return total[0, 0]


# speedup vs baseline: 3.3266x; 1.0201x over previous
import functools

import jax
import jax.numpy as jnp
from jax import lax
from jax.experimental import pallas as pl
from jax.experimental.pallas import tpu as pltpu

B = 16384
C = 1000
MARGIN = 1.0

BR = 2048
GRID = B // BR


def _tc_hinge_body(x_ref, y_ref, out_ref):
    pi = pl.program_id(0)
    x = x_ref[...]
    yv = y_ref[0, 0, :]
    ycol = yv.reshape(BR, 1)
    col = lax.broadcasted_iota(jnp.int32, (BR, C), 1)
    oy = jnp.sum(jnp.where(col == ycol, x, 0.0), axis=1, keepdims=True)
    s = jnp.sum(jnp.maximum(x - oy + MARGIN, 0.0))

    @pl.when(pi == 0)
    def _init():
        out_ref[0, 0] = 0.0

    out_ref[0, 0] += s

    @pl.when(pi == GRID - 1)
    def _final():
        out_ref[0, 0] = (out_ref[0, 0] - float(B)) / float(B)


_tc_hinge = pl.pallas_call(
    _tc_hinge_body,
    grid=(GRID,),
    in_specs=[
        pl.BlockSpec((BR, C), lambda i: (i, 0)),
        pl.BlockSpec((1, 1, BR), lambda i: (i, 0, 0)),
    ],
    out_specs=pl.BlockSpec((1, 1), lambda i: (0, 0), memory_space=pltpu.SMEM),
    out_shape=jax.ShapeDtypeStruct((1, 1), jnp.float32),
)


def kernel(output, y):
    y3 = y.astype(jnp.int32).reshape(GRID, 1, BR)
    total = _tc_hinge(output, y3)
    return total[0, 0]
